# Initial kernel scaffold; baseline (speedup 1.0000x reference)
import jax
import jax.numpy as jnp
from jax.experimental import pallas as pl


def kernel(user_id, pos_id, neg_id, user_emb, item_emb, edge_row, edge_col, edge_val):
    raise NotImplementedError("write your pallas kernel here")


# speedup vs baseline: 4.3374x; 4.3374x over previous
import functools

import jax
import jax.numpy as jnp
from jax import lax
from jax.experimental import pallas as pl
from jax.experimental.pallas import tpu as pltpu
from jax.experimental.pallas import tpu_sc as plsc

NUM_USER = 5000
NUM_ITEM = 5000
D = 128
NN = NUM_USER + NUM_ITEM
N_LAYERS = 3
N_EDGES = 320000
B = 4096
N_NEG = 8

NC = 2
NS = 16
NW = NC * NS

NN_PAD = 10240
K = 128
EPT = 10240
E_PAD = EPT * NW
N_CHUNK = EPT // K
RPT = NN_PAD // NS
UPT = B // NW
G = 16


@functools.cache
def _make_propagate():
    mesh = plsc.VectorSubcoreMesh(core_axis_name="c", subcore_axis_name="s",
                                  num_cores=NC, num_subcores=NS)

    @functools.partial(
        pl.kernel,
        out_type=jax.ShapeDtypeStruct((NC, NN_PAD, D), jnp.float32),
        mesh=mesh,
        scratch_types=[
            pltpu.VMEM((K,), jnp.int32),
            pltpu.VMEM((K,), jnp.int32),
            pltpu.VMEM((K,), jnp.float32),
            pltpu.VMEM((K, D), jnp.float32),
            pltpu.VMEM_SHARED((NN_PAD, D), jnp.float32),
        ],
    )
    def _propagate(t_hbm, row_hbm, col_hbm, val_hbm, p_hbm,
                   colv, rowv, valv, rows, acc):
        cid = lax.axis_index("c")
        sid = lax.axis_index("s")
        wid = cid * NS + sid

        zero16 = jnp.zeros((16,), jnp.float32)

        def _zrow(i, carry):
            for k in range(D // 16):
                rows[i, pl.ds(k * 16, 16)] = zero16
            return carry

        lax.fori_loop(0, K, _zrow, 0)
        for j in range(RPT // K):
            pltpu.sync_copy(rows, acc.at[pl.ds(sid * RPT + j * K, K)])
        plsc.subcore_barrier()

        def _chunk(ci, carry):
            base = wid * EPT + ci * K
            pltpu.sync_copy(col_hbm.at[pl.ds(base, K)], colv)
            pltpu.sync_copy(row_hbm.at[pl.ds(base, K)], rowv)
            pltpu.sync_copy(val_hbm.at[pl.ds(base, K)], valv)
            pltpu.sync_copy(t_hbm.at[colv], rows)

            def _scale(g, c2):
                vals = valv[pl.ds(g * 16, 16)]
                for j in range(16):
                    e = g * 16 + j
                    v = vals[j]
                    for k in range(D // 16):
                        rows[e, pl.ds(k * 16, 16)] = (
                            rows[e, pl.ds(k * 16, 16)] * v)
                return c2

            lax.fori_loop(0, K // 16, _scale, 0)
            pltpu.sync_copy(rows, acc.at[rowv], add=True)
            return carry

        lax.fori_loop(0, N_CHUNK, _chunk, 0)
        plsc.subcore_barrier()

        pltpu.sync_copy(acc.at[pl.ds(sid * RPT, RPT)],
                        p_hbm.at[cid, pl.ds(sid * RPT, RPT)])

    return _propagate


def _combine_body(p_ref, s_ref, t_ref, snew_ref):
    t = p_ref[0] + p_ref[1]
    t_ref[...] = t
    snew_ref[...] = s_ref[...] + t


_R = 1024


def _combine(p, s):
    return pl.pallas_call(
        _combine_body,
        grid=(NN_PAD // _R,),
        in_specs=[
            pl.BlockSpec((NC, _R, D), lambda i: (0, i, 0)),
            pl.BlockSpec((_R, D), lambda i: (i, 0)),
        ],
        out_specs=[
            pl.BlockSpec((_R, D), lambda i: (i, 0)),
            pl.BlockSpec((_R, D), lambda i: (i, 0)),
        ],
        out_shape=[
            jax.ShapeDtypeStruct((NN_PAD, D), jnp.float32),
            jax.ShapeDtypeStruct((NN_PAD, D), jnp.float32),
        ],
    )(p, s)


@functools.cache
def _make_gather_rows():
    mesh = plsc.VectorSubcoreMesh(core_axis_name="c", subcore_axis_name="s",
                                  num_cores=NC, num_subcores=NS)

    @functools.partial(
        pl.kernel,
        out_type=(
            jax.ShapeDtypeStruct((B, D), jnp.float32),
            jax.ShapeDtypeStruct((B, D), jnp.float32),
            jax.ShapeDtypeStruct((B * N_NEG, D), jnp.float32),
        ),
        mesh=mesh,
        scratch_types=[
            pltpu.VMEM((UPT,), jnp.int32),
            pltpu.VMEM((UPT, D), jnp.float32),
        ],
    )
    def _gather_rows(s_hbm, uid_hbm, pid_hbm, nid_hbm,
                     u_hbm, p_hbm, n_hbm, idxv, rowsv):
        cid = lax.axis_index("c")
        sid = lax.axis_index("s")
        wid = cid * NS + sid
        u0 = wid * UPT
        off = jnp.full((16,), NUM_USER, jnp.int32)

        pltpu.sync_copy(uid_hbm.at[pl.ds(u0, UPT)], idxv)
        pltpu.sync_copy(s_hbm.at[idxv], rowsv)
        pltpu.sync_copy(rowsv, u_hbm.at[pl.ds(u0, UPT)])
        pltpu.sync_copy(pid_hbm.at[pl.ds(u0, UPT)], idxv)
        for k in range(UPT // 16):
            idxv[pl.ds(k * 16, 16)] = idxv[pl.ds(k * 16, 16)] + off
        pltpu.sync_copy(s_hbm.at[idxv], rowsv)
        pltpu.sync_copy(rowsv, p_hbm.at[pl.ds(u0, UPT)])

        def _negchunk(ci, carry):
            n0 = u0 * N_NEG + ci * UPT
            pltpu.sync_copy(nid_hbm.at[pl.ds(n0, UPT)], idxv)
            for k in range(UPT // 16):
                idxv[pl.ds(k * 16, 16)] = idxv[pl.ds(k * 16, 16)] + off
            pltpu.sync_copy(s_hbm.at[idxv], rowsv)
            pltpu.sync_copy(rowsv, n_hbm.at[pl.ds(n0, UPT)])
            return carry

        lax.fori_loop(0, N_NEG, _negchunk, 0)

    return _gather_rows


_BB = 512


def _dots_body(u_ref, p_ref, n_ref, pos_ref, neg_ref):
    u = u_ref[...]
    pos_ref[...] = (jnp.sum(u * p_ref[...], axis=1) * 0.0625)[None, :]
    neg_ref[...] = jnp.sum(u[:, None, :] * n_ref[...], axis=2) * 0.0625


def _dots(u, p, n3):
    return pl.pallas_call(
        _dots_body,
        grid=(B // _BB,),
        in_specs=[
            pl.BlockSpec((_BB, D), lambda i: (i, 0)),
            pl.BlockSpec((_BB, D), lambda i: (i, 0)),
            pl.BlockSpec((_BB, N_NEG, D), lambda i: (i, 0, 0)),
        ],
        out_specs=[
            pl.BlockSpec((1, _BB), lambda i: (0, i)),
            pl.BlockSpec((_BB, N_NEG), lambda i: (i, 0)),
        ],
        out_shape=[
            jax.ShapeDtypeStruct((1, B), jnp.float32),
            jax.ShapeDtypeStruct((B, N_NEG), jnp.float32),
        ],
    )(u, p, n3)


def kernel(user_id, pos_id, neg_id, user_emb, item_emb,
           edge_row, edge_col, edge_val):
    t0 = jnp.concatenate([
        user_emb, item_emb,
        jnp.zeros((NN_PAD - NN, D), jnp.float32)], axis=0)

    pad = E_PAD - N_EDGES
    pad_idx = (jnp.arange(pad, dtype=jnp.int32) * 37) % NN
    row_p = jnp.concatenate([edge_row, pad_idx])
    col_p = jnp.concatenate([edge_col, pad_idx])
    val_p = jnp.concatenate([edge_val, jnp.zeros((pad,), jnp.float32)])

    propagate = _make_propagate()
    gather_rows = _make_gather_rows()

    s = t0
    t = t0
    for _ in range(N_LAYERS):
        p = propagate(t, row_p, col_p, val_p)
        t, s = _combine(p, s)

    neg_flat = neg_id.reshape(-1)
    u_rows, p_rows, n_rows = gather_rows(s, user_id, pos_id, neg_flat)
    pos2, neg_rat = _dots(u_rows, p_rows, n_rows.reshape(B, N_NEG, D))
    return (pos2.reshape(B), neg_rat)


# speedup vs baseline: 9.3303x; 2.1511x over previous
import functools

import jax
import jax.numpy as jnp
from jax import lax
from jax.experimental import pallas as pl
from jax.experimental.pallas import tpu as pltpu
from jax.experimental.pallas import tpu_sc as plsc

NUM_USER = 5000
NUM_ITEM = 5000
D = 128
NN = NUM_USER + NUM_ITEM
N_LAYERS = 3
N_EDGES = 320000
B = 4096
N_NEG = 8

NC = 2
NS = 16
NW = NC * NS

NN_PAD = 10240
K = 128
EPT = 10240
E_PAD = EPT * NW
N_CHUNK = EPT // K
RPT = NN_PAD // NS
UPT = B // NW
G = 16


@functools.cache
def _make_propagate():
    mesh = plsc.VectorSubcoreMesh(core_axis_name="c", subcore_axis_name="s",
                                  num_cores=NC, num_subcores=NS)

    @functools.partial(
        pl.kernel,
        out_type=jax.ShapeDtypeStruct((NC, NN_PAD, D), jnp.float32),
        mesh=mesh,
        scratch_types=[
            [pltpu.VMEM((K,), jnp.int32) for _ in range(4)],
            [pltpu.VMEM((K,), jnp.int32) for _ in range(4)],
            [pltpu.VMEM((K,), jnp.float32) for _ in range(4)],
            [pltpu.VMEM((K, D), jnp.float32) for _ in range(2)],
            [pltpu.SemaphoreType.DMA for _ in range(4)],
            [pltpu.SemaphoreType.DMA for _ in range(2)],
            [pltpu.SemaphoreType.DMA for _ in range(2)],
            pltpu.VMEM_SHARED((NN_PAD, D), jnp.float32),
        ],
    )
    def _propagate(t_hbm, row_hbm, col_hbm, val_hbm, p_hbm,
                   colb, rowb, valb, rbs, isems, gsems, ssems, acc):
        cid = lax.axis_index("c")
        sid = lax.axis_index("s")
        wid = cid * NS + sid
        c0 = wid * N_CHUNK

        def _start_idx(ci, sl):
            pltpu.async_copy(col_hbm.at[c0 + ci], colb[sl], isems[sl])
            pltpu.async_copy(row_hbm.at[c0 + ci], rowb[sl], isems[sl])
            pltpu.async_copy(val_hbm.at[c0 + ci], valb[sl], isems[sl])

        def _wait_idx(ci, sl):
            pltpu.make_async_copy(col_hbm.at[c0 + ci], colb[sl],
                                  isems[sl]).wait()
            pltpu.make_async_copy(row_hbm.at[c0 + ci], rowb[sl],
                                  isems[sl]).wait()
            pltpu.make_async_copy(val_hbm.at[c0 + ci], valb[sl],
                                  isems[sl]).wait()

        def _start_gather(sl, b):
            pltpu.async_copy(t_hbm.at[colb[sl]], rbs[b], gsems[b])

        def _wait_gather(sl, b):
            pltpu.make_async_copy(t_hbm.at[colb[sl]], rbs[b],
                                  gsems[b]).wait()

        def _start_scatter(sl, b):
            pltpu.async_copy(rbs[b], acc.at[rowb[sl]], ssems[b], add=True)

        def _wait_scatter(sl, b):
            pltpu.make_async_copy(rbs[b], acc.at[rowb[sl]],
                                  ssems[b]).wait()

        def _scale(sl, b):
            rb = rbs[b]

            def _sg(g, c2):
                vals = valb[sl][pl.ds(g * 16, 16)]
                for j in range(16):
                    e = g * 16 + j
                    v = vals[j]
                    for k in range(D // 16):
                        rb[e, pl.ds(k * 16, 16)] = (
                            rb[e, pl.ds(k * 16, 16)] * v)
                return c2

            lax.fori_loop(0, K // 16, _sg, 0)

        _start_idx(0, 0)
        _start_idx(1, 1)

        zero16 = jnp.zeros((16,), jnp.float32)

        def _zrow(i, carry):
            for k in range(D // 16):
                rbs[0][i, pl.ds(k * 16, 16)] = zero16
            return carry

        lax.fori_loop(0, K, _zrow, 0)
        for j in range(RPT // K):
            pltpu.sync_copy(rbs[0], acc.at[pl.ds(sid * RPT + j * K, K)])
        plsc.subcore_barrier()

        _wait_idx(0, 0)
        _start_gather(0, 0)

        def _quad(m, carry):
            for cp in range(4):
                c = m * 4 + cp
                b = cp & 1
                sl = cp
                sl_n = (cp + 1) & 3
                sl_p = (cp + 2) & 3

                @pl.when(c + 2 < N_CHUNK)
                def _():
                    _start_idx(c + 2, sl_p)

                _wait_gather(sl, b)

                @pl.when(c >= 1)
                def _():
                    _wait_scatter((cp + 3) & 3, 1 - b)

                @pl.when(c + 1 < N_CHUNK)
                def _():
                    _wait_idx(c + 1, sl_n)
                    _start_gather(sl_n, 1 - b)

                _scale(sl, b)
                _start_scatter(sl, b)
            return carry

        lax.fori_loop(0, N_CHUNK // 4, _quad, 0)
        _wait_scatter(3, 1)
        plsc.subcore_barrier()

        pltpu.sync_copy(acc.at[pl.ds(sid * RPT, RPT)],
                        p_hbm.at[cid, pl.ds(sid * RPT, RPT)])

    return _propagate


def _combine_body(p_ref, s_ref, t_ref, snew_ref):
    t = p_ref[0] + p_ref[1]
    t_ref[...] = t
    snew_ref[...] = s_ref[...] + t


_R = 1024


def _combine(p, s):
    return pl.pallas_call(
        _combine_body,
        grid=(NN_PAD // _R,),
        in_specs=[
            pl.BlockSpec((NC, _R, D), lambda i: (0, i, 0)),
            pl.BlockSpec((_R, D), lambda i: (i, 0)),
        ],
        out_specs=[
            pl.BlockSpec((_R, D), lambda i: (i, 0)),
            pl.BlockSpec((_R, D), lambda i: (i, 0)),
        ],
        out_shape=[
            jax.ShapeDtypeStruct((NN_PAD, D), jnp.float32),
            jax.ShapeDtypeStruct((NN_PAD, D), jnp.float32),
        ],
    )(p, s)


@functools.cache
def _make_gather_rows():
    mesh = plsc.VectorSubcoreMesh(core_axis_name="c", subcore_axis_name="s",
                                  num_cores=NC, num_subcores=NS)

    @functools.partial(
        pl.kernel,
        out_type=(
            jax.ShapeDtypeStruct((B, D), jnp.float32),
            jax.ShapeDtypeStruct((B, D), jnp.float32),
            jax.ShapeDtypeStruct((B * N_NEG, D), jnp.float32),
        ),
        mesh=mesh,
        scratch_types=[
            pltpu.VMEM((UPT,), jnp.int32),
            pltpu.VMEM((UPT, D), jnp.float32),
        ],
    )
    def _gather_rows(s_hbm, uid_hbm, pid_hbm, nid_hbm,
                     u_hbm, p_hbm, n_hbm, idxv, rowsv):
        cid = lax.axis_index("c")
        sid = lax.axis_index("s")
        wid = cid * NS + sid
        u0 = wid * UPT
        off = jnp.full((16,), NUM_USER, jnp.int32)

        pltpu.sync_copy(uid_hbm.at[pl.ds(u0, UPT)], idxv)
        pltpu.sync_copy(s_hbm.at[idxv], rowsv)
        pltpu.sync_copy(rowsv, u_hbm.at[pl.ds(u0, UPT)])
        pltpu.sync_copy(pid_hbm.at[pl.ds(u0, UPT)], idxv)
        for k in range(UPT // 16):
            idxv[pl.ds(k * 16, 16)] = idxv[pl.ds(k * 16, 16)] + off
        pltpu.sync_copy(s_hbm.at[idxv], rowsv)
        pltpu.sync_copy(rowsv, p_hbm.at[pl.ds(u0, UPT)])

        def _negchunk(ci, carry):
            n0 = u0 * N_NEG + ci * UPT
            pltpu.sync_copy(nid_hbm.at[pl.ds(n0, UPT)], idxv)
            for k in range(UPT // 16):
                idxv[pl.ds(k * 16, 16)] = idxv[pl.ds(k * 16, 16)] + off
            pltpu.sync_copy(s_hbm.at[idxv], rowsv)
            pltpu.sync_copy(rowsv, n_hbm.at[pl.ds(n0, UPT)])
            return carry

        lax.fori_loop(0, N_NEG, _negchunk, 0)

    return _gather_rows


_BB = 512


def _dots_body(u_ref, p_ref, n_ref, pos_ref, neg_ref):
    u = u_ref[...]
    pos_ref[...] = (jnp.sum(u * p_ref[...], axis=1) * 0.0625)[None, :]
    neg_ref[...] = jnp.sum(u[:, None, :] * n_ref[...], axis=2) * 0.0625


def _dots(u, p, n3):
    return pl.pallas_call(
        _dots_body,
        grid=(B // _BB,),
        in_specs=[
            pl.BlockSpec((_BB, D), lambda i: (i, 0)),
            pl.BlockSpec((_BB, D), lambda i: (i, 0)),
            pl.BlockSpec((_BB, N_NEG, D), lambda i: (i, 0, 0)),
        ],
        out_specs=[
            pl.BlockSpec((1, _BB), lambda i: (0, i)),
            pl.BlockSpec((_BB, N_NEG), lambda i: (i, 0)),
        ],
        out_shape=[
            jax.ShapeDtypeStruct((1, B), jnp.float32),
            jax.ShapeDtypeStruct((B, N_NEG), jnp.float32),
        ],
    )(u, p, n3)


def kernel(user_id, pos_id, neg_id, user_emb, item_emb,
           edge_row, edge_col, edge_val):
    t0 = jnp.concatenate([
        user_emb, item_emb,
        jnp.zeros((NN_PAD - NN, D), jnp.float32)], axis=0)

    pad = E_PAD - N_EDGES
    pad_idx = (jnp.arange(pad, dtype=jnp.int32) * 37) % NN
    row_p = jnp.concatenate([edge_row, pad_idx]).reshape(E_PAD // K, K)
    col_p = jnp.concatenate([edge_col, pad_idx]).reshape(E_PAD // K, K)
    val_p = jnp.concatenate(
        [edge_val, jnp.zeros((pad,), jnp.float32)]).reshape(E_PAD // K, K)

    propagate = _make_propagate()
    gather_rows = _make_gather_rows()

    s = t0
    t = t0
    for _ in range(N_LAYERS):
        p = propagate(t, row_p, col_p, val_p)
        t, s = _combine(p, s)

    neg_flat = neg_id.reshape(-1)
    u_rows, p_rows, n_rows = gather_rows(s, user_id, pos_id, neg_flat)
    pos2, neg_rat = _dots(u_rows, p_rows, n_rows.reshape(B, N_NEG, D))
    return (pos2.reshape(B), neg_rat)


# speedup vs baseline: 9.5345x; 1.0219x over previous
import functools

import jax
import jax.numpy as jnp
from jax import lax
from jax.experimental import pallas as pl
from jax.experimental.pallas import tpu as pltpu
from jax.experimental.pallas import tpu_sc as plsc

NUM_USER = 5000
NUM_ITEM = 5000
D = 128
NN = NUM_USER + NUM_ITEM
N_LAYERS = 3
N_EDGES = 320000
B = 4096
N_NEG = 8

NC = 2
NS = 16
NW = NC * NS

NN_PAD = 10240
K = 128
EPT = 10240
E_PAD = EPT * NW
N_CHUNK = EPT // K
RPT = NN_PAD // NS
UPT = B // NW
G = 16


@functools.cache
def _make_propagate():
    mesh = plsc.VectorSubcoreMesh(core_axis_name="c", subcore_axis_name="s",
                                  num_cores=NC, num_subcores=NS)

    @functools.partial(
        pl.kernel,
        out_type=jax.ShapeDtypeStruct((NC, NN_PAD, D), jnp.float32),
        mesh=mesh,
        scratch_types=[
            [pltpu.VMEM((K,), jnp.int32) for _ in range(4)],
            [pltpu.VMEM((K,), jnp.int32) for _ in range(4)],
            [pltpu.VMEM((K,), jnp.float32) for _ in range(4)],
            [pltpu.VMEM((K, D), jnp.float32) for _ in range(2)],
            [pltpu.SemaphoreType.DMA for _ in range(4)],
            [pltpu.SemaphoreType.DMA for _ in range(2)],
            [pltpu.SemaphoreType.DMA for _ in range(2)],
            pltpu.VMEM_SHARED((NN_PAD, D), jnp.float32),
        ],
    )
    def _propagate(t_hbm, row_hbm, col_hbm, val_hbm, p_hbm,
                   colb, rowb, valb, rbs, isems, gsems, ssems, acc):
        cid = lax.axis_index("c")
        sid = lax.axis_index("s")
        wid = cid * NS + sid
        c0 = wid * N_CHUNK

        def _start_idx(ci, sl):
            pltpu.async_copy(col_hbm.at[c0 + ci], colb[sl], isems[sl])
            pltpu.async_copy(row_hbm.at[c0 + ci], rowb[sl], isems[sl])
            pltpu.async_copy(val_hbm.at[c0 + ci], valb[sl], isems[sl])

        def _wait_idx(ci, sl):
            pltpu.make_async_copy(col_hbm.at[c0 + ci], colb[sl],
                                  isems[sl]).wait()
            pltpu.make_async_copy(row_hbm.at[c0 + ci], rowb[sl],
                                  isems[sl]).wait()
            pltpu.make_async_copy(val_hbm.at[c0 + ci], valb[sl],
                                  isems[sl]).wait()

        def _start_gather(sl, b):
            pltpu.async_copy(t_hbm.at[colb[sl]], rbs[b], gsems[b])

        def _wait_gather(sl, b):
            pltpu.make_async_copy(t_hbm.at[colb[sl]], rbs[b],
                                  gsems[b]).wait()

        def _start_scatter(sl, b):
            pltpu.async_copy(rbs[b], acc.at[rowb[sl]], ssems[b], add=True)

        def _wait_scatter(sl, b):
            pltpu.make_async_copy(rbs[b], acc.at[rowb[sl]],
                                  ssems[b]).wait()

        def _scale(sl, b):
            rb = rbs[b]

            def _sg(g, c2):
                vals = valb[sl][pl.ds(g * 16, 16)]
                for j in range(16):
                    e = g * 16 + j
                    v = vals[j]
                    for k in range(D // 16):
                        rb[e, pl.ds(k * 16, 16)] = (
                            rb[e, pl.ds(k * 16, 16)] * v)
                return c2

            lax.fori_loop(0, K // 16, _sg, 0)

        _start_idx(0, 0)
        _start_idx(1, 1)

        zero16 = jnp.zeros((16,), jnp.float32)

        def _zrow(i, carry):
            for k in range(D // 16):
                rbs[0][i, pl.ds(k * 16, 16)] = zero16
            return carry

        lax.fori_loop(0, K, _zrow, 0)
        for j in range(RPT // K):
            pltpu.sync_copy(rbs[0], acc.at[pl.ds(sid * RPT + j * K, K)])
        plsc.subcore_barrier()

        _wait_idx(0, 0)
        _start_gather(0, 0)

        def _quad(m, carry):
            for cp in range(4):
                c = m * 4 + cp
                b = cp & 1
                sl = cp
                sl_n = (cp + 1) & 3
                sl_p = (cp + 2) & 3

                @pl.when(c + 2 < N_CHUNK)
                def _():
                    _start_idx(c + 2, sl_p)

                @pl.when(c >= 1)
                def _():
                    _wait_scatter((cp + 3) & 3, 1 - b)

                @pl.when(c + 1 < N_CHUNK)
                def _():
                    _wait_idx(c + 1, sl_n)
                    _start_gather(sl_n, 1 - b)

                _wait_gather(sl, b)
                _scale(sl, b)
                _start_scatter(sl, b)
            return carry

        lax.fori_loop(0, N_CHUNK // 4, _quad, 0)
        _wait_scatter(3, 1)
        plsc.subcore_barrier()

        pltpu.sync_copy(acc.at[pl.ds(sid * RPT, RPT)],
                        p_hbm.at[cid, pl.ds(sid * RPT, RPT)])

    return _propagate


def _combine_body(p_ref, s_ref, t_ref, snew_ref):
    t = p_ref[0] + p_ref[1]
    t_ref[...] = t
    snew_ref[...] = s_ref[...] + t


_R = 1024


def _combine(p, s):
    return pl.pallas_call(
        _combine_body,
        grid=(NN_PAD // _R,),
        in_specs=[
            pl.BlockSpec((NC, _R, D), lambda i: (0, i, 0)),
            pl.BlockSpec((_R, D), lambda i: (i, 0)),
        ],
        out_specs=[
            pl.BlockSpec((_R, D), lambda i: (i, 0)),
            pl.BlockSpec((_R, D), lambda i: (i, 0)),
        ],
        out_shape=[
            jax.ShapeDtypeStruct((NN_PAD, D), jnp.float32),
            jax.ShapeDtypeStruct((NN_PAD, D), jnp.float32),
        ],
    )(p, s)


@functools.cache
def _make_gather_rows():
    mesh = plsc.VectorSubcoreMesh(core_axis_name="c", subcore_axis_name="s",
                                  num_cores=NC, num_subcores=NS)

    @functools.partial(
        pl.kernel,
        out_type=(
            jax.ShapeDtypeStruct((B, D), jnp.float32),
            jax.ShapeDtypeStruct((B, D), jnp.float32),
            jax.ShapeDtypeStruct((B * N_NEG, D), jnp.float32),
        ),
        mesh=mesh,
        scratch_types=[
            pltpu.VMEM((UPT,), jnp.int32),
            pltpu.VMEM((UPT, D), jnp.float32),
        ],
    )
    def _gather_rows(s_hbm, uid_hbm, pid_hbm, nid_hbm,
                     u_hbm, p_hbm, n_hbm, idxv, rowsv):
        cid = lax.axis_index("c")
        sid = lax.axis_index("s")
        wid = cid * NS + sid
        u0 = wid * UPT
        off = jnp.full((16,), NUM_USER, jnp.int32)

        pltpu.sync_copy(uid_hbm.at[pl.ds(u0, UPT)], idxv)
        pltpu.sync_copy(s_hbm.at[idxv], rowsv)
        pltpu.sync_copy(rowsv, u_hbm.at[pl.ds(u0, UPT)])
        pltpu.sync_copy(pid_hbm.at[pl.ds(u0, UPT)], idxv)
        for k in range(UPT // 16):
            idxv[pl.ds(k * 16, 16)] = idxv[pl.ds(k * 16, 16)] + off
        pltpu.sync_copy(s_hbm.at[idxv], rowsv)
        pltpu.sync_copy(rowsv, p_hbm.at[pl.ds(u0, UPT)])

        def _negchunk(ci, carry):
            n0 = u0 * N_NEG + ci * UPT
            pltpu.sync_copy(nid_hbm.at[pl.ds(n0, UPT)], idxv)
            for k in range(UPT // 16):
                idxv[pl.ds(k * 16, 16)] = idxv[pl.ds(k * 16, 16)] + off
            pltpu.sync_copy(s_hbm.at[idxv], rowsv)
            pltpu.sync_copy(rowsv, n_hbm.at[pl.ds(n0, UPT)])
            return carry

        lax.fori_loop(0, N_NEG, _negchunk, 0)

    return _gather_rows


_BB = 512


def _dots_body(u_ref, p_ref, n_ref, pos_ref, neg_ref):
    u = u_ref[...]
    pos_ref[...] = (jnp.sum(u * p_ref[...], axis=1) * 0.0625)[None, :]
    neg_ref[...] = jnp.sum(u[:, None, :] * n_ref[...], axis=2) * 0.0625


def _dots(u, p, n3):
    return pl.pallas_call(
        _dots_body,
        grid=(B // _BB,),
        in_specs=[
            pl.BlockSpec((_BB, D), lambda i: (i, 0)),
            pl.BlockSpec((_BB, D), lambda i: (i, 0)),
            pl.BlockSpec((_BB, N_NEG, D), lambda i: (i, 0, 0)),
        ],
        out_specs=[
            pl.BlockSpec((1, _BB), lambda i: (0, i)),
            pl.BlockSpec((_BB, N_NEG), lambda i: (i, 0)),
        ],
        out_shape=[
            jax.ShapeDtypeStruct((1, B), jnp.float32),
            jax.ShapeDtypeStruct((B, N_NEG), jnp.float32),
        ],
    )(u, p, n3)


def kernel(user_id, pos_id, neg_id, user_emb, item_emb,
           edge_row, edge_col, edge_val):
    t0 = jnp.concatenate([
        user_emb, item_emb,
        jnp.zeros((NN_PAD - NN, D), jnp.float32)], axis=0)

    pad = E_PAD - N_EDGES
    pad_idx = (jnp.arange(pad, dtype=jnp.int32) * 37) % NN
    row_p = jnp.concatenate([edge_row, pad_idx]).reshape(E_PAD // K, K)
    col_p = jnp.concatenate([edge_col, pad_idx]).reshape(E_PAD // K, K)
    val_p = jnp.concatenate(
        [edge_val, jnp.zeros((pad,), jnp.float32)]).reshape(E_PAD // K, K)

    propagate = _make_propagate()
    gather_rows = _make_gather_rows()

    s = t0
    t = t0
    for _ in range(N_LAYERS):
        p = propagate(t, row_p, col_p, val_p)
        t, s = _combine(p, s)

    neg_flat = neg_id.reshape(-1)
    u_rows, p_rows, n_rows = gather_rows(s, user_id, pos_id, neg_flat)
    pos2, neg_rat = _dots(u_rows, p_rows, n_rows.reshape(B, N_NEG, D))
    return (pos2.reshape(B), neg_rat)


# speedup vs baseline: 10.2907x; 1.0793x over previous
import functools

import jax
import jax.numpy as jnp
from jax import lax
from jax.experimental import pallas as pl
from jax.experimental.pallas import tpu as pltpu
from jax.experimental.pallas import tpu_sc as plsc

NUM_USER = 5000
NUM_ITEM = 5000
D = 128
NN = NUM_USER + NUM_ITEM
N_LAYERS = 3
N_EDGES = 320000
B = 4096
N_NEG = 8

NC = 2
NS = 16
NW = NC * NS

NN_PAD = 10240
K = 80
EPT = 10240
E_PAD = EPT * NW
N_CHUNK = EPT // K
RPT = NN_PAD // NS
UPT = B // NW
G = 16


@functools.cache
def _make_propagate():
    mesh = plsc.VectorSubcoreMesh(core_axis_name="c", subcore_axis_name="s",
                                  num_cores=NC, num_subcores=NS)

    @functools.partial(
        pl.kernel,
        out_type=jax.ShapeDtypeStruct((NC, NN_PAD, D), jnp.float32),
        mesh=mesh,
        scratch_types=[
            [pltpu.VMEM((K,), jnp.int32) for _ in range(8)],
            [pltpu.VMEM((K,), jnp.int32) for _ in range(8)],
            [pltpu.VMEM((K,), jnp.float32) for _ in range(8)],
            [pltpu.VMEM((K, D), jnp.float32) for _ in range(4)],
            [pltpu.SemaphoreType.DMA for _ in range(8)],
            [pltpu.SemaphoreType.DMA for _ in range(4)],
            [pltpu.SemaphoreType.DMA for _ in range(4)],
            pltpu.VMEM_SHARED((NN_PAD, D), jnp.float32),
        ],
    )
    def _propagate(t_hbm, row_hbm, col_hbm, val_hbm, p_hbm,
                   colb, rowb, valb, rbs, isems, gsems, ssems, acc):
        cid = lax.axis_index("c")
        sid = lax.axis_index("s")
        wid = cid * NS + sid
        c0 = wid * N_CHUNK

        def _start_idx(ci, sl):
            pltpu.async_copy(col_hbm.at[c0 + ci], colb[sl], isems[sl])
            pltpu.async_copy(row_hbm.at[c0 + ci], rowb[sl], isems[sl])
            pltpu.async_copy(val_hbm.at[c0 + ci], valb[sl], isems[sl])

        def _wait_idx(ci, sl):
            pltpu.make_async_copy(col_hbm.at[c0 + ci], colb[sl],
                                  isems[sl]).wait()
            pltpu.make_async_copy(row_hbm.at[c0 + ci], rowb[sl],
                                  isems[sl]).wait()
            pltpu.make_async_copy(val_hbm.at[c0 + ci], valb[sl],
                                  isems[sl]).wait()

        def _start_gather(sl, b):
            pltpu.async_copy(t_hbm.at[colb[sl]], rbs[b], gsems[b])

        def _wait_gather(sl, b):
            pltpu.make_async_copy(t_hbm.at[colb[sl]], rbs[b],
                                  gsems[b]).wait()

        def _start_scatter(sl, b):
            pltpu.async_copy(rbs[b], acc.at[rowb[sl]], ssems[b], add=True)

        def _wait_scatter(sl, b):
            pltpu.make_async_copy(rbs[b], acc.at[rowb[sl]],
                                  ssems[b]).wait()

        def _scale(sl, b):
            rb = rbs[b]

            def _sg(g, c2):
                vals = valb[sl][pl.ds(g * 16, 16)]
                for j in range(16):
                    e = g * 16 + j
                    v = vals[j]
                    for k in range(D // 16):
                        rb[e, pl.ds(k * 16, 16)] = (
                            rb[e, pl.ds(k * 16, 16)] * v)
                return c2

            lax.fori_loop(0, K // 16, _sg, 0)

        _start_idx(0, 0)
        _start_idx(1, 1)

        zero16 = jnp.zeros((16,), jnp.float32)

        def _zrow(i, carry):
            for k in range(D // 16):
                rbs[0][i, pl.ds(k * 16, 16)] = zero16
            return carry

        lax.fori_loop(0, K, _zrow, 0)
        for j in range(RPT // K):
            pltpu.sync_copy(rbs[0], acc.at[pl.ds(sid * RPT + j * K, K)])
        plsc.subcore_barrier()

        for ci in range(2, 4):
            _start_idx(ci, ci)
        _wait_idx(0, 0)
        _start_gather(0, 0)
        _wait_idx(1, 1)
        _start_gather(1, 1)

        def _oct(m, carry):
            for cp in range(8):
                c = m * 8 + cp
                b = cp & 3
                b2 = (cp + 2) & 3
                sl = cp
                sl_p = (cp + 4) & 7
                sl_n2 = (cp + 2) & 7

                @pl.when(c + 4 < N_CHUNK)
                def _():
                    _start_idx(c + 4, sl_p)

                @pl.when(c >= 2)
                def _():
                    _wait_scatter((cp + 6) & 7, b2)

                @pl.when(c + 2 < N_CHUNK)
                def _():
                    _wait_idx(c + 2, sl_n2)
                    _start_gather(sl_n2, b2)

                _wait_gather(sl, b)
                _scale(sl, b)
                _start_scatter(sl, b)
            return carry

        lax.fori_loop(0, N_CHUNK // 8, _oct, 0)
        _wait_scatter(6, 2)
        _wait_scatter(7, 3)
        plsc.subcore_barrier()

        pltpu.sync_copy(acc.at[pl.ds(sid * RPT, RPT)],
                        p_hbm.at[cid, pl.ds(sid * RPT, RPT)])

    return _propagate


def _combine_body(p_ref, s_ref, t_ref, snew_ref):
    t = p_ref[0] + p_ref[1]
    t_ref[...] = t
    snew_ref[...] = s_ref[...] + t


_R = 1024


def _combine(p, s):
    return pl.pallas_call(
        _combine_body,
        grid=(NN_PAD // _R,),
        in_specs=[
            pl.BlockSpec((NC, _R, D), lambda i: (0, i, 0)),
            pl.BlockSpec((_R, D), lambda i: (i, 0)),
        ],
        out_specs=[
            pl.BlockSpec((_R, D), lambda i: (i, 0)),
            pl.BlockSpec((_R, D), lambda i: (i, 0)),
        ],
        out_shape=[
            jax.ShapeDtypeStruct((NN_PAD, D), jnp.float32),
            jax.ShapeDtypeStruct((NN_PAD, D), jnp.float32),
        ],
    )(p, s)


@functools.cache
def _make_gather_rows():
    mesh = plsc.VectorSubcoreMesh(core_axis_name="c", subcore_axis_name="s",
                                  num_cores=NC, num_subcores=NS)

    @functools.partial(
        pl.kernel,
        out_type=(
            jax.ShapeDtypeStruct((B, D), jnp.float32),
            jax.ShapeDtypeStruct((B, D), jnp.float32),
            jax.ShapeDtypeStruct((B * N_NEG, D), jnp.float32),
        ),
        mesh=mesh,
        scratch_types=[
            pltpu.VMEM((UPT,), jnp.int32),
            pltpu.VMEM((UPT, D), jnp.float32),
        ],
    )
    def _gather_rows(s_hbm, uid_hbm, pid_hbm, nid_hbm,
                     u_hbm, p_hbm, n_hbm, idxv, rowsv):
        cid = lax.axis_index("c")
        sid = lax.axis_index("s")
        wid = cid * NS + sid
        u0 = wid * UPT
        off = jnp.full((16,), NUM_USER, jnp.int32)

        pltpu.sync_copy(uid_hbm.at[pl.ds(u0, UPT)], idxv)
        pltpu.sync_copy(s_hbm.at[idxv], rowsv)
        pltpu.sync_copy(rowsv, u_hbm.at[pl.ds(u0, UPT)])
        pltpu.sync_copy(pid_hbm.at[pl.ds(u0, UPT)], idxv)
        for k in range(UPT // 16):
            idxv[pl.ds(k * 16, 16)] = idxv[pl.ds(k * 16, 16)] + off
        pltpu.sync_copy(s_hbm.at[idxv], rowsv)
        pltpu.sync_copy(rowsv, p_hbm.at[pl.ds(u0, UPT)])

        def _negchunk(ci, carry):
            n0 = u0 * N_NEG + ci * UPT
            pltpu.sync_copy(nid_hbm.at[pl.ds(n0, UPT)], idxv)
            for k in range(UPT // 16):
                idxv[pl.ds(k * 16, 16)] = idxv[pl.ds(k * 16, 16)] + off
            pltpu.sync_copy(s_hbm.at[idxv], rowsv)
            pltpu.sync_copy(rowsv, n_hbm.at[pl.ds(n0, UPT)])
            return carry

        lax.fori_loop(0, N_NEG, _negchunk, 0)

    return _gather_rows


_BB = 512


def _dots_body(u_ref, p_ref, n_ref, pos_ref, neg_ref):
    u = u_ref[...]
    pos_ref[...] = (jnp.sum(u * p_ref[...], axis=1) * 0.0625)[None, :]
    neg_ref[...] = jnp.sum(u[:, None, :] * n_ref[...], axis=2) * 0.0625


def _dots(u, p, n3):
    return pl.pallas_call(
        _dots_body,
        grid=(B // _BB,),
        in_specs=[
            pl.BlockSpec((_BB, D), lambda i: (i, 0)),
            pl.BlockSpec((_BB, D), lambda i: (i, 0)),
            pl.BlockSpec((_BB, N_NEG, D), lambda i: (i, 0, 0)),
        ],
        out_specs=[
            pl.BlockSpec((1, _BB), lambda i: (0, i)),
            pl.BlockSpec((_BB, N_NEG), lambda i: (i, 0)),
        ],
        out_shape=[
            jax.ShapeDtypeStruct((1, B), jnp.float32),
            jax.ShapeDtypeStruct((B, N_NEG), jnp.float32),
        ],
    )(u, p, n3)


def kernel(user_id, pos_id, neg_id, user_emb, item_emb,
           edge_row, edge_col, edge_val):
    t0 = jnp.concatenate([
        user_emb, item_emb,
        jnp.zeros((NN_PAD - NN, D), jnp.float32)], axis=0)

    pad = E_PAD - N_EDGES
    pad_idx = (jnp.arange(pad, dtype=jnp.int32) * 37) % NN
    row_p = jnp.concatenate([edge_row, pad_idx]).reshape(E_PAD // K, K)
    col_p = jnp.concatenate([edge_col, pad_idx]).reshape(E_PAD // K, K)
    val_p = jnp.concatenate(
        [edge_val, jnp.zeros((pad,), jnp.float32)]).reshape(E_PAD // K, K)

    propagate = _make_propagate()
    gather_rows = _make_gather_rows()

    s = t0
    t = t0
    for _ in range(N_LAYERS):
        p = propagate(t, row_p, col_p, val_p)
        t, s = _combine(p, s)

    neg_flat = neg_id.reshape(-1)
    u_rows, p_rows, n_rows = gather_rows(s, user_id, pos_id, neg_flat)
    pos2, neg_rat = _dots(u_rows, p_rows, n_rows.reshape(B, N_NEG, D))
    return (pos2.reshape(B), neg_rat)


# speedup vs baseline: 10.4678x; 1.0172x over previous
import functools

import jax
import jax.numpy as jnp
from jax import lax
from jax.experimental import pallas as pl
from jax.experimental.pallas import tpu as pltpu
from jax.experimental.pallas import tpu_sc as plsc

NUM_USER = 5000
NUM_ITEM = 5000
D = 128
NN = NUM_USER + NUM_ITEM
N_LAYERS = 3
N_EDGES = 320000
B = 4096
N_NEG = 8

NC = 2
NS = 16
NW = NC * NS

NN_PAD = 10240
K = 80
EPT = 10240
E_PAD = EPT * NW
N_CHUNK = EPT // K
RPT = NN_PAD // NS
UPT = B // NW
G = 16


@functools.cache
def _make_propagate():
    mesh = plsc.VectorSubcoreMesh(core_axis_name="c", subcore_axis_name="s",
                                  num_cores=NC, num_subcores=NS)

    @functools.partial(
        pl.kernel,
        out_type=jax.ShapeDtypeStruct((NC, NN_PAD, D), jnp.float32),
        mesh=mesh,
        scratch_types=[
            [pltpu.VMEM((K,), jnp.int32) for _ in range(8)],
            [pltpu.VMEM((K,), jnp.int32) for _ in range(8)],
            [pltpu.VMEM((K,), jnp.float32) for _ in range(8)],
            [pltpu.VMEM((K, D), jnp.float32) for _ in range(4)],
            [pltpu.SemaphoreType.DMA for _ in range(8)],
            [pltpu.SemaphoreType.DMA for _ in range(4)],
            [pltpu.SemaphoreType.DMA for _ in range(4)],
            pltpu.VMEM_SHARED((NN_PAD, D), jnp.float32),
        ],
    )
    def _propagate(t_hbm, row_hbm, col_hbm, val_hbm, p_hbm,
                   colb, rowb, valb, rbs, isems, gsems, ssems, acc):
        cid = lax.axis_index("c")
        sid = lax.axis_index("s")
        wid = cid * NS + sid
        c0 = wid * N_CHUNK

        def _start_idx(ci, sl):
            pltpu.async_copy(col_hbm.at[c0 + ci], colb[sl], isems[sl])
            pltpu.async_copy(row_hbm.at[c0 + ci], rowb[sl], isems[sl])
            pltpu.async_copy(val_hbm.at[c0 + ci], valb[sl], isems[sl])

        def _wait_idx(ci, sl):
            pltpu.make_async_copy(col_hbm.at[c0 + ci], colb[sl],
                                  isems[sl]).wait()
            pltpu.make_async_copy(row_hbm.at[c0 + ci], rowb[sl],
                                  isems[sl]).wait()
            pltpu.make_async_copy(val_hbm.at[c0 + ci], valb[sl],
                                  isems[sl]).wait()

        def _start_gather(sl, b):
            pltpu.async_copy(t_hbm.at[colb[sl]], rbs[b], gsems[b])

        def _wait_gather(sl, b):
            pltpu.make_async_copy(t_hbm.at[colb[sl]], rbs[b],
                                  gsems[b]).wait()

        def _start_scatter(sl, b):
            pltpu.async_copy(rbs[b], acc.at[rowb[sl]], ssems[b], add=True)

        def _wait_scatter(sl, b):
            pltpu.make_async_copy(rbs[b], acc.at[rowb[sl]],
                                  ssems[b]).wait()

        def _scale(sl, b):
            rb = rbs[b]

            def _sg(g, c2):
                vals = valb[sl][pl.ds(g * 16, 16)]
                for j in range(16):
                    e = g * 16 + j
                    v = vals[j]
                    for k in range(D // 16):
                        rb[e, pl.ds(k * 16, 16)] = (
                            rb[e, pl.ds(k * 16, 16)] * v)
                return c2

            lax.fori_loop(0, K // 16, _sg, 0)

        _start_idx(0, 0)
        _start_idx(1, 1)

        zero16 = jnp.zeros((16,), jnp.float32)

        def _zrow(i, carry):
            for k in range(D // 16):
                rbs[0][i, pl.ds(k * 16, 16)] = zero16
            return carry

        lax.fori_loop(0, K, _zrow, 0)
        for j in range(RPT // K):
            pltpu.sync_copy(rbs[0], acc.at[pl.ds(sid * RPT + j * K, K)])
        plsc.subcore_barrier()

        for ci in range(2, 4):
            _start_idx(ci, ci)
        _wait_idx(0, 0)
        _start_gather(0, 0)
        _wait_idx(1, 1)
        _start_gather(1, 1)

        def _oct(m, carry):
            for cp in range(8):
                c = m * 8 + cp
                b = cp & 3
                b2 = (cp + 2) & 3
                sl = cp
                sl_p = (cp + 4) & 7
                sl_n2 = (cp + 2) & 7

                @pl.when(c + 4 < N_CHUNK)
                def _():
                    _start_idx(c + 4, sl_p)

                @pl.when(c >= 2)
                def _():
                    _wait_scatter((cp + 6) & 7, b2)

                @pl.when(c + 2 < N_CHUNK)
                def _():
                    _wait_idx(c + 2, sl_n2)
                    _start_gather(sl_n2, b2)

                _wait_gather(sl, b)
                _scale(sl, b)
                _start_scatter(sl, b)
            return carry

        lax.fori_loop(0, N_CHUNK // 8, _oct, 0)
        _wait_scatter(6, 2)
        _wait_scatter(7, 3)
        plsc.subcore_barrier()

        pltpu.sync_copy(acc.at[pl.ds(sid * RPT, RPT)],
                        p_hbm.at[cid, pl.ds(sid * RPT, RPT)])

    return _propagate


def _combine_body(p_ref, s_ref, t_ref, snew_ref):
    t = p_ref[0] + p_ref[1]
    t_ref[...] = t
    snew_ref[...] = s_ref[...] + t


_R = 1024


def _combine(p, s):
    return pl.pallas_call(
        _combine_body,
        grid=(NN_PAD // _R,),
        in_specs=[
            pl.BlockSpec((NC, _R, D), lambda i: (0, i, 0)),
            pl.BlockSpec((_R, D), lambda i: (i, 0)),
        ],
        out_specs=[
            pl.BlockSpec((_R, D), lambda i: (i, 0)),
            pl.BlockSpec((_R, D), lambda i: (i, 0)),
        ],
        out_shape=[
            jax.ShapeDtypeStruct((NN_PAD, D), jnp.float32),
            jax.ShapeDtypeStruct((NN_PAD, D), jnp.float32),
        ],
    )(p, s)


@functools.cache
def _make_gather_rows():
    mesh = plsc.VectorSubcoreMesh(core_axis_name="c", subcore_axis_name="s",
                                  num_cores=NC, num_subcores=NS)

    @functools.partial(
        pl.kernel,
        out_type=(
            jax.ShapeDtypeStruct((B, D), jnp.float32),
            jax.ShapeDtypeStruct((B, D), jnp.float32),
            jax.ShapeDtypeStruct((B * N_NEG, D), jnp.float32),
        ),
        mesh=mesh,
        scratch_types=[
            [pltpu.VMEM((UPT,), jnp.int32) for _ in range(2)],
            [pltpu.VMEM((UPT, D), jnp.float32) for _ in range(2)],
            [pltpu.SemaphoreType.DMA for _ in range(2)],
            [pltpu.SemaphoreType.DMA for _ in range(2)],
        ],
    )
    def _gather_rows(s_hbm, uid_hbm, pid_hbm, nid_hbm,
                     u_hbm, p_hbm, n_hbm, idxbs, rowsbs, gsems, osems):
        cid = lax.axis_index("c")
        sid = lax.axis_index("s")
        wid = cid * NS + sid
        u0 = wid * UPT
        off = jnp.full((16,), NUM_USER, jnp.int32)
        n_st = 2 + N_NEG

        def _prep(j):
            ib = idxbs[j & 1]
            if j == 0:
                pltpu.sync_copy(uid_hbm.at[pl.ds(u0, UPT)], ib)
            else:
                if j == 1:
                    pltpu.sync_copy(pid_hbm.at[pl.ds(u0, UPT)], ib)
                else:
                    n0 = u0 * N_NEG + (j - 2) * UPT
                    pltpu.sync_copy(nid_hbm.at[pl.ds(n0, UPT)], ib)
                for k in range(UPT // 16):
                    ib[pl.ds(k * 16, 16)] = ib[pl.ds(k * 16, 16)] + off

        def _dst(j):
            if j == 0:
                return u_hbm.at[pl.ds(u0, UPT)]
            if j == 1:
                return p_hbm.at[pl.ds(u0, UPT)]
            return n_hbm.at[pl.ds(u0 * N_NEG + (j - 2) * UPT, UPT)]

        def _start_gather(j):
            pltpu.async_copy(s_hbm.at[idxbs[j & 1]], rowsbs[j & 1],
                             gsems[j & 1])

        def _wait_gather(j):
            pltpu.make_async_copy(s_hbm.at[idxbs[j & 1]], rowsbs[j & 1],
                                  gsems[j & 1]).wait()

        _prep(0)
        _start_gather(0)
        _prep(1)
        for j in range(n_st):
            b = j & 1
            if j + 1 < n_st:
                if j >= 1:
                    pltpu.make_async_copy(rowsbs[1 - b], _dst(j - 1),
                                          osems[1 - b]).wait()
                _start_gather(j + 1)
            _wait_gather(j)
            if j + 2 < n_st:
                _prep(j + 2)
            pltpu.async_copy(rowsbs[b], _dst(j), osems[b])
        pltpu.make_async_copy(rowsbs[0], _dst(n_st - 2), osems[0]).wait()
        pltpu.make_async_copy(rowsbs[1], _dst(n_st - 1), osems[1]).wait()

    return _gather_rows


_BB = 512


def _dots_body(u_ref, p_ref, n_ref, pos_ref, neg_ref):
    u = u_ref[...]
    pos_ref[...] = (jnp.sum(u * p_ref[...], axis=1) * 0.0625)[None, :]
    neg_ref[...] = jnp.sum(u[:, None, :] * n_ref[...], axis=2) * 0.0625


def _dots(u, p, n3):
    return pl.pallas_call(
        _dots_body,
        grid=(B // _BB,),
        in_specs=[
            pl.BlockSpec((_BB, D), lambda i: (i, 0)),
            pl.BlockSpec((_BB, D), lambda i: (i, 0)),
            pl.BlockSpec((_BB, N_NEG, D), lambda i: (i, 0, 0)),
        ],
        out_specs=[
            pl.BlockSpec((1, _BB), lambda i: (0, i)),
            pl.BlockSpec((_BB, N_NEG), lambda i: (i, 0)),
        ],
        out_shape=[
            jax.ShapeDtypeStruct((1, B), jnp.float32),
            jax.ShapeDtypeStruct((B, N_NEG), jnp.float32),
        ],
    )(u, p, n3)


def kernel(user_id, pos_id, neg_id, user_emb, item_emb,
           edge_row, edge_col, edge_val):
    t0 = jnp.concatenate([
        user_emb, item_emb,
        jnp.zeros((NN_PAD - NN, D), jnp.float32)], axis=0)

    pad = E_PAD - N_EDGES
    pad_idx = (jnp.arange(pad, dtype=jnp.int32) * 37) % NN
    row_p = jnp.concatenate([edge_row, pad_idx]).reshape(E_PAD // K, K)
    col_p = jnp.concatenate([edge_col, pad_idx]).reshape(E_PAD // K, K)
    val_p = jnp.concatenate(
        [edge_val, jnp.zeros((pad,), jnp.float32)]).reshape(E_PAD // K, K)

    propagate = _make_propagate()
    gather_rows = _make_gather_rows()

    s = t0
    t = t0
    for _ in range(N_LAYERS):
        p = propagate(t, row_p, col_p, val_p)
        t, s = _combine(p, s)

    neg_flat = neg_id.reshape(-1)
    u_rows, p_rows, n_rows = gather_rows(s, user_id, pos_id, neg_flat)
    pos2, neg_rat = _dots(u_rows, p_rows, n_rows.reshape(B, N_NEG, D))
    return (pos2.reshape(B), neg_rat)
